# baseline (speedup 1.0000x reference)
import functools

import numpy as np
import jax
import jax.numpy as jnp
from jax import lax
from jax.experimental import pallas as pl
from jax.experimental.pallas import tpu as pltpu
from jax.experimental.pallas import tpu_sc as plsc

H, W = 1024, 1024
OD = (500, 1500)
HP, WP = 5, 5
THR_VAR = 100.0
THETA_MIN = -np.pi / 2.0
THETA_MAX = np.pi / 2.0
RHO_MAX = float(np.sqrt(H ** 2 + W ** 2))
RHO_MIN = -RHO_MAX
TB = (THETA_MAX - THETA_MIN) / (OD[0] - 1)
RB = (RHO_MAX - RHO_MIN) / (OD[1] - 1)

_xs = np.arange(WP, dtype=np.float32) - (WP - 1) / 2.0
SXX = float(HP * np.sum(_xs ** 2))
SYY = float(WP * np.sum(_xs ** 2))
N = float(HP * WP)

NBINS = OD[0] * OD[1]
NB_PAD = 750080
NPIX = H * W

_NT = 16
_ROWS = NPIX // 128
_TROWS = _ROWS // _NT
_CROWS = 32
_NCHUNK = _TROWS // _CROWS
_SEG = NB_PAD // _NT


_F = np.float32
C_1_50 = _F(1.0) / _F(50.0)
C_1_25 = _F(1.0) / _F(25.0)
C_1_22 = _F(1.0) / _F(22.0)
C_TB = _F(TB)
C_INV_TB = _F(1.0) / _F(TB)
C_RB = _F(RB)
C_INV_RB = _F(1.0) / _F(RB)
C_RHOMAX = _F(RHO_MAX)
C_PI = _F(np.pi)
C_HALF_PI = _F(np.pi / 2.0)
C_TBRB = _F(C_TB * C_RB)
C_2PI = _F(2.0 * np.pi)


def _tc_body(s1_ref, sx_ref, sy_ref, s2_ref, mask_ref, idx_ref, w_ref):
    s1 = s1_ref[...]
    sx = sx_ref[...]
    sy = sy_ref[...]
    s2 = s2_ref[...]
    mask = mask_ref[...]

    alpha = sx * C_1_50
    beta = sy * C_1_50
    gamma = s1 * C_1_25
    s2m = s2 - (gamma * gamma) * _F(25.0)

    t1 = jnp.arctan2(beta, alpha) + C_HALF_PI
    r = lax.rem(t1, C_PI)
    tm = jnp.where((r < _F(0.0)) & (r != _F(0.0)), r + C_PI, r)
    theta = tm + (-C_HALF_PI)

    a2 = alpha * alpha
    b2 = beta * beta
    denom = jnp.maximum(_F(1e-12), a2 + b2)
    dta = (-beta) / denom
    dtb = alpha / denom
    sumsq = dta * dta + dtb * dtb
    ss_res = (s2m - a2 * _F(50.0)) - b2 * _F(50.0)
    noise_var = jnp.maximum(_F(1e-6), ss_res * C_1_22)
    va = noise_var * C_1_50
    var_theta = sumsq * va

    st_ = jnp.sin(theta)
    ct_ = jnp.cos(theta)
    y = lax.broadcasted_iota(jnp.int32, (H, W), 0).astype(jnp.float32)
    x = lax.broadcasted_iota(jnp.int32, (H, W), 1).astype(jnp.float32)

    drdt = (-x) * st_ + y * ct_
    var_rho = (drdt * drdt) * var_theta + _F(1.0)

    idx_t = lax.convert_element_type(
        jnp.clip(jnp.floor(tm * C_INV_TB), _F(0.0), _F(499.0)), jnp.int32)
    ctb = idx_t.astype(jnp.float32) * C_TB + (-C_HALF_PI)
    d1 = theta - ctb

    rho = x * ct_ + y * st_
    idx_r = lax.convert_element_type(
        jnp.clip(jnp.floor((rho + C_RHOMAX) * C_INV_RB), _F(0.0), _F(1499.0)),
        jnp.int32)
    crb = idx_r.astype(jnp.float32) * C_RB + (-C_RHOMAX)
    d2 = rho - crb

    covar = drdt * var_theta
    num = (var_rho * (d1 * d1) - ((covar * _F(2.0)) * d1) * d2) \
        + var_theta * (d2 * d2)
    num = jnp.maximum(num, _F(0.0))
    det = jnp.maximum(_F(1e-12), var_theta * var_rho - covar * covar)
    q = num / det
    pdf = jnp.exp(q * _F(-0.5)) / (jnp.sqrt(det) * C_2PI)
    w = jnp.where(var_theta <= _F(100.0), pdf * C_TBRB, _F(0.0)) * mask

    idx_ref[...] = idx_t * OD[1] + idx_r
    w_ref[...] = w


@functools.cache
def _make_sc_scatter():
    mesh = plsc.VectorSubcoreMesh(
        core_axis_name="c", subcore_axis_name="s", num_cores=1)

    @functools.partial(
        pl.kernel,
        mesh=mesh,
        out_type=jax.ShapeDtypeStruct((NB_PAD,), jnp.float32),
        scratch_types=[
            pltpu.VMEM((2, _CROWS, 128), jnp.int32),
            pltpu.VMEM((2, _CROWS, 128), jnp.float32),
            pltpu.VMEM((4096,), jnp.float32),
            pltpu.VMEM_SHARED((NB_PAD,), jnp.float32),
            pltpu.SemaphoreType.DMA,
            pltpu.SemaphoreType.DMA,
            pltpu.SemaphoreType.DMA,
        ],
    )
    def _sc_scatter(idx_hbm, w_hbm, out_hbm, idx_v, w_v, zbuf, acc_sh,
                    sem0, sem1, sem_sc):
        wid = lax.axis_index("s")
        seg = wid * _SEG
        sems = (sem0, sem1)

        def zstep(i, carry):
            for u in range(8):
                zbuf[pl.ds((i * 8 + u) * 16, 16)] = jnp.zeros((16,), jnp.float32)
            return carry
        lax.fori_loop(0, 4096 // 128, zstep, 0)
        for k in range(11):
            pltpu.sync_copy(zbuf, acc_sh.at[pl.ds(seg + k * 4096, 4096)])
        pltpu.sync_copy(zbuf.at[pl.ds(0, 1824)],
                        acc_sh.at[pl.ds(seg + 45056, 1824)])
        plsc.subcore_barrier()

        row0 = wid * _TROWS
        pltpu.async_copy(idx_hbm.at[pl.ds(row0, _CROWS)], idx_v.at[0], sem0)
        pltpu.async_copy(w_hbm.at[pl.ds(row0, _CROWS)], w_v.at[0], sem0)

        def pair(i, carry):
            for p in (0, 1):
                g = i * 2 + p
                r_cur = row0 + g * _CROWS
                pltpu.make_async_copy(
                    idx_hbm.at[pl.ds(r_cur, _CROWS)], idx_v.at[p], sems[p]).wait()
                pltpu.make_async_copy(
                    w_hbm.at[pl.ds(r_cur, _CROWS)], w_v.at[p], sems[p]).wait()

                @pl.when(g + 1 < _NCHUNK)
                def _():
                    r_nxt = row0 + (g + 1) * _CROWS
                    pltpu.async_copy(idx_hbm.at[pl.ds(r_nxt, _CROWS)],
                                     idx_v.at[1 - p], sems[1 - p])
                    pltpu.async_copy(w_hbm.at[pl.ds(r_nxt, _CROWS)],
                                     w_v.at[1 - p], sems[1 - p])

                descs = [
                    pltpu.async_copy(
                        w_v.at[p, j], acc_sh.at[idx_v.at[p, j]], sem_sc,
                        add=True)
                    for j in range(_CROWS)
                ]
                for dsc in descs:
                    dsc.wait()
            return carry

        lax.fori_loop(0, _NCHUNK // 2, pair, 0)
        plsc.subcore_barrier()

        for k in range(11):
            pltpu.sync_copy(acc_sh.at[pl.ds(seg + k * 4096, 4096)], zbuf)
            pltpu.sync_copy(zbuf, out_hbm.at[pl.ds(seg + k * 4096, 4096)])
        pltpu.sync_copy(acc_sh.at[pl.ds(seg + 45056, 1824)],
                        zbuf.at[pl.ds(0, 1824)])
        pltpu.sync_copy(zbuf.at[pl.ds(0, 1824)],
                        out_hbm.at[pl.ds(seg + 45056, 1824)])

    return _sc_scatter


def _conv2(img, ker):
    return jax.lax.conv_general_dilated(
        img[None, None], ker[None, None].astype(img.dtype), (1, 1), 'SAME')[0, 0]


def kernel(img, mask):
    xs = np.arange(WP, dtype=np.float32) - (WP - 1) / 2.0
    ys = np.arange(HP, dtype=np.float32) - (HP - 1) / 2.0
    Kx = jnp.asarray(np.tile(xs[None, :], (HP, 1)))
    Ky = jnp.asarray(np.tile(ys[:, None], (1, WP)))
    Kones = jnp.ones((HP, WP), jnp.float32)
    s1 = _conv2(img, Kones)
    sx = _conv2(img, Kx)
    sy = _conv2(img, Ky)
    s2 = _conv2(img * img, Kones)
    idx, w = pl.pallas_call(
        _tc_body,
        out_shape=(
            jax.ShapeDtypeStruct((H, W), jnp.int32),
            jax.ShapeDtypeStruct((H, W), jnp.float32),
        ),
    )(s1, sx, sy, s2, mask)
    acc = _make_sc_scatter()(idx.reshape(_ROWS, 128), w.reshape(_ROWS, 128))
    return acc[:NBINS].reshape(OD)


# speedup vs baseline: 3.5045x; 3.5045x over previous
import functools

import numpy as np
import jax
import jax.numpy as jnp
from jax import lax
from jax.experimental import pallas as pl
from jax.experimental.pallas import tpu as pltpu
from jax.experimental.pallas import tpu_sc as plsc

H, W = 1024, 1024
OD = (500, 1500)
HP, WP = 5, 5
THR_VAR = 100.0
THETA_MIN = -np.pi / 2.0
THETA_MAX = np.pi / 2.0
RHO_MAX = float(np.sqrt(H ** 2 + W ** 2))
RHO_MIN = -RHO_MAX
TB = (THETA_MAX - THETA_MIN) / (OD[0] - 1)
RB = (RHO_MAX - RHO_MIN) / (OD[1] - 1)

_xs = np.arange(WP, dtype=np.float32) - (WP - 1) / 2.0
SXX = float(HP * np.sum(_xs ** 2))
SYY = float(WP * np.sum(_xs ** 2))
N = float(HP * WP)

NBINS = OD[0] * OD[1]
NB_PAD = 750080
NPIX = H * W

_NT = 16
_ROWS = NPIX // 128
_TROWS = _ROWS // _NT
_CROWS = 32
_NCHUNK = _TROWS // _CROWS
_SEG = NB_PAD // _NT


_F = np.float32
C_1_50 = _F(1.0) / _F(50.0)
C_1_25 = _F(1.0) / _F(25.0)
C_1_22 = _F(1.0) / _F(22.0)
C_TB = _F(TB)
C_INV_TB = _F(1.0) / _F(TB)
C_RB = _F(RB)
C_INV_RB = _F(1.0) / _F(RB)
C_RHOMAX = _F(RHO_MAX)
C_PI = _F(np.pi)
C_HALF_PI = _F(np.pi / 2.0)
C_TBRB = _F(C_TB * C_RB)
C_2PI = _F(2.0 * np.pi)


def _tc_body(s1_ref, sx_ref, sy_ref, s2_ref, mask_ref, idx_ref, w_ref):
    s1 = s1_ref[...]
    sx = sx_ref[...]
    sy = sy_ref[...]
    s2 = s2_ref[...]
    mask = mask_ref[...]

    alpha = sx * C_1_50
    beta = sy * C_1_50
    gamma = s1 * C_1_25
    s2m = s2 - (gamma * gamma) * _F(25.0)

    t1 = jnp.arctan2(beta, alpha) + C_HALF_PI
    r = lax.rem(t1, C_PI)
    tm = jnp.where((r < _F(0.0)) & (r != _F(0.0)), r + C_PI, r)
    theta = tm + (-C_HALF_PI)

    a2 = alpha * alpha
    b2 = beta * beta
    denom = jnp.maximum(_F(1e-12), a2 + b2)
    dta = (-beta) / denom
    dtb = alpha / denom
    sumsq = dta * dta + dtb * dtb
    ss_res = (s2m - a2 * _F(50.0)) - b2 * _F(50.0)
    noise_var = jnp.maximum(_F(1e-6), ss_res * C_1_22)
    va = noise_var * C_1_50
    var_theta = sumsq * va

    st_ = jnp.sin(theta)
    ct_ = jnp.cos(theta)
    x = lax.broadcasted_iota(jnp.int32, (W, H), 0).astype(jnp.float32)
    y = lax.broadcasted_iota(jnp.int32, (W, H), 1).astype(jnp.float32)

    drdt = (-x) * st_ + y * ct_
    var_rho = (drdt * drdt) * var_theta + _F(1.0)

    idx_t = lax.convert_element_type(
        jnp.clip(jnp.floor(tm * C_INV_TB), _F(0.0), _F(499.0)), jnp.int32)
    ctb = idx_t.astype(jnp.float32) * C_TB + (-C_HALF_PI)
    d1 = theta - ctb

    rho = x * ct_ + y * st_
    idx_r = lax.convert_element_type(
        jnp.clip(jnp.floor((rho + C_RHOMAX) * C_INV_RB), _F(0.0), _F(1499.0)),
        jnp.int32)
    crb = idx_r.astype(jnp.float32) * C_RB + (-C_RHOMAX)
    d2 = rho - crb

    covar = drdt * var_theta
    num = (var_rho * (d1 * d1) - ((covar * _F(2.0)) * d1) * d2) \
        + var_theta * (d2 * d2)
    num = jnp.maximum(num, _F(0.0))
    det = jnp.maximum(_F(1e-12), var_theta * var_rho - covar * covar)
    q = num / det
    pdf = jnp.exp(q * _F(-0.5)) / (jnp.sqrt(det) * C_2PI)
    w = jnp.where(var_theta <= _F(100.0), pdf * C_TBRB, _F(0.0)) * mask

    idx_ref[...] = idx_t * OD[1] + idx_r
    w_ref[...] = w


@functools.cache
def _make_sc_scatter():
    mesh = plsc.VectorSubcoreMesh(
        core_axis_name="c", subcore_axis_name="s", num_cores=1)

    @functools.partial(
        pl.kernel,
        mesh=mesh,
        out_type=jax.ShapeDtypeStruct((NB_PAD,), jnp.float32),
        scratch_types=[
            pltpu.VMEM((2, _CROWS, 128), jnp.int32),
            pltpu.VMEM((2, _CROWS, 128), jnp.float32),
            pltpu.VMEM((4096,), jnp.float32),
            pltpu.VMEM_SHARED((NB_PAD,), jnp.float32),
            pltpu.SemaphoreType.DMA,
            pltpu.SemaphoreType.DMA,
            pltpu.SemaphoreType.DMA,
        ],
    )
    def _sc_scatter(idx_hbm, w_hbm, out_hbm, idx_v, w_v, zbuf, acc_sh,
                    sem0, sem1, sem_sc):
        wid = lax.axis_index("s")
        seg = wid * _SEG
        sems = (sem0, sem1)

        def zstep(i, carry):
            for u in range(8):
                zbuf[pl.ds((i * 8 + u) * 16, 16)] = jnp.zeros((16,), jnp.float32)
            return carry
        lax.fori_loop(0, 4096 // 128, zstep, 0)
        for k in range(11):
            pltpu.sync_copy(zbuf, acc_sh.at[pl.ds(seg + k * 4096, 4096)])
        pltpu.sync_copy(zbuf.at[pl.ds(0, 1824)],
                        acc_sh.at[pl.ds(seg + 45056, 1824)])
        plsc.subcore_barrier()

        row0 = wid * _TROWS
        pltpu.async_copy(idx_hbm.at[pl.ds(row0, _CROWS)], idx_v.at[0], sem0)
        pltpu.async_copy(w_hbm.at[pl.ds(row0, _CROWS)], w_v.at[0], sem0)

        def pair(i, carry):
            for p in (0, 1):
                g = i * 2 + p
                r_cur = row0 + g * _CROWS
                pltpu.make_async_copy(
                    idx_hbm.at[pl.ds(r_cur, _CROWS)], idx_v.at[p], sems[p]).wait()
                pltpu.make_async_copy(
                    w_hbm.at[pl.ds(r_cur, _CROWS)], w_v.at[p], sems[p]).wait()

                @pl.when(g + 1 < _NCHUNK)
                def _():
                    r_nxt = row0 + (g + 1) * _CROWS
                    pltpu.async_copy(idx_hbm.at[pl.ds(r_nxt, _CROWS)],
                                     idx_v.at[1 - p], sems[1 - p])
                    pltpu.async_copy(w_hbm.at[pl.ds(r_nxt, _CROWS)],
                                     w_v.at[1 - p], sems[1 - p])

                descs = [
                    pltpu.async_copy(
                        w_v.at[p, j], acc_sh.at[idx_v.at[p, j]], sem_sc,
                        add=True)
                    for j in range(_CROWS)
                ]
                for dsc in descs:
                    dsc.wait()
            return carry

        lax.fori_loop(0, _NCHUNK // 2, pair, 0)
        plsc.subcore_barrier()

        for k in range(11):
            pltpu.sync_copy(acc_sh.at[pl.ds(seg + k * 4096, 4096)], zbuf)
            pltpu.sync_copy(zbuf, out_hbm.at[pl.ds(seg + k * 4096, 4096)])
        pltpu.sync_copy(acc_sh.at[pl.ds(seg + 45056, 1824)],
                        zbuf.at[pl.ds(0, 1824)])
        pltpu.sync_copy(zbuf.at[pl.ds(0, 1824)],
                        out_hbm.at[pl.ds(seg + 45056, 1824)])

    return _sc_scatter


def _conv2(img, ker):
    return jax.lax.conv_general_dilated(
        img[None, None], ker[None, None].astype(img.dtype), (1, 1), 'SAME')[0, 0]


def kernel(img, mask):
    xs = np.arange(WP, dtype=np.float32) - (WP - 1) / 2.0
    ys = np.arange(HP, dtype=np.float32) - (HP - 1) / 2.0
    Kx = jnp.asarray(np.tile(xs[None, :], (HP, 1)))
    Ky = jnp.asarray(np.tile(ys[:, None], (1, WP)))
    Kones = jnp.ones((HP, WP), jnp.float32)
    s1 = _conv2(img, Kones)
    sx = _conv2(img, Kx)
    sy = _conv2(img, Ky)
    s2 = _conv2(img * img, Kones)
    idx, w = pl.pallas_call(
        _tc_body,
        out_shape=(
            jax.ShapeDtypeStruct((W, H), jnp.int32),
            jax.ShapeDtypeStruct((W, H), jnp.float32),
        ),
    )(s1.T, sx.T, sy.T, s2.T, mask.T)
    acc = _make_sc_scatter()(idx.reshape(_ROWS, 128), w.reshape(_ROWS, 128))
    return acc[:NBINS].reshape(OD)
